# baseline (speedup 1.0000x reference)
import functools

import jax
import jax.numpy as jnp
from jax import lax
from jax.experimental import pallas as pl
from jax.experimental.pallas import tpu as pltpu
from jax.experimental.pallas import tpu_sc as plsc

DIM = 64
B = 16384
NC = 2
NS = 16
NW = NC * NS
BPW = B // NW
CHUNK = 128
NCHUNK = BPW // CHUNK
LANES = 16
DCH = DIM // LANES


def _sc_body(center_hbm, context_hbm, table_hbm, out_hbm,
             idx_c, idx_x, rows_c, rows_x, outv, sem):
    c = lax.axis_index("c")
    s = lax.axis_index("s")
    wid = s * NC + c
    base = wid * NCHUNK

    pltpu.sync_copy(center_hbm.at[pl.ds(base, NCHUNK)], idx_c)
    pltpu.sync_copy(context_hbm.at[pl.ds(base, NCHUNK)], idx_x)

    copies = []
    for j in range(NCHUNK):
        copies.append(pltpu.async_copy(
            table_hbm.at[idx_c.at[j]], rows_c.at[pl.ds(j * CHUNK, CHUNK)], sem))
        copies.append(pltpu.async_copy(
            table_hbm.at[idx_x.at[j]], rows_x.at[pl.ds(j * CHUNK, CHUNK)], sem))
    for cp in copies:
        cp.wait()

    def body(i, accs):
        return tuple(
            accs[d] + rows_c[i, pl.ds(d * LANES, LANES)]
            * rows_x[i, pl.ds(d * LANES, LANES)]
            for d in range(DCH))

    accs = lax.fori_loop(
        0, BPW, body,
        tuple(jnp.zeros((LANES,), jnp.float32) for _ in range(DCH)))
    outv[...] = (accs[0] + accs[1]) + (accs[2] + accs[3])
    pltpu.sync_copy(outv, out_hbm.at[wid])


_sc_call = functools.partial(
    pl.kernel,
    mesh=plsc.VectorSubcoreMesh(core_axis_name="c", subcore_axis_name="s"),
    out_type=jax.ShapeDtypeStruct((NW, LANES), jnp.float32),
    scratch_types=[
        pltpu.VMEM((NCHUNK, CHUNK), jnp.int32),
        pltpu.VMEM((NCHUNK, CHUNK), jnp.int32),
        pltpu.VMEM((BPW, DIM), jnp.float32),
        pltpu.VMEM((BPW, DIM), jnp.float32),
        pltpu.VMEM((LANES,), jnp.float32),
        pltpu.SemaphoreType.DMA,
    ],
    compiler_params=pltpu.CompilerParams(use_tc_tiling_on_sc=False),
)(_sc_body)


@jax.jit
def kernel(center_words, context_words, embeddings):
    c2 = jnp.asarray(center_words, jnp.int32).reshape(NW * NCHUNK, CHUNK)
    x2 = jnp.asarray(context_words, jnp.int32).reshape(NW * NCHUNK, CHUNK)
    partials = _sc_call(c2, x2, embeddings)
    return jnp.sum(partials)
